# Initial kernel scaffold; baseline (speedup 1.0000x reference)
import jax
import jax.numpy as jnp
from jax.experimental import pallas as pl


def kernel(u, v, r, support, support_t, u_side, v_side, u_features, v_features, W_gc, W_u1, b_u1, W_v1, b_v1, W_u2, W_v2, W_bil, a_mix):
    raise NotImplementedError("write your pallas kernel here")


# speedup vs baseline: 1.5503x; 1.5503x over previous
import functools

import jax
import jax.numpy as jnp
from jax import lax
from jax.experimental import pallas as pl
from jax.experimental.pallas import tpu as pltpu
from jax.experimental.pallas import tpu_sc as plsc

F32 = jnp.float32

NU = 2048
NI = 1536
NC = 5
FD = NU + NI
H0 = 64
H1 = 32
ID = 10
NBASIS = 2
NE = 100000

E_PAD = 100352
KB = 512


def _feat_kernel(x_ref, w_ref, out_ref):
    k = pl.program_id(0)

    @pl.when(k == 0)
    def _():
        out_ref[...] = jnp.zeros_like(out_ref)

    out_ref[...] += jnp.dot(x_ref[...], w_ref[...], preferred_element_type=F32)

    @pl.when(k == pl.num_programs(0) - 1)
    def _():
        for c in range(1, NC):
            out_ref[:, c * H0:(c + 1) * H0] += out_ref[:, (c - 1) * H0:c * H0]


def _feat_matmul(x, w_flat):
    m = x.shape[0]
    return pl.pallas_call(
        _feat_kernel,
        grid=(FD // KB,),
        in_specs=[
            pl.BlockSpec((m, KB), lambda k: (0, k)),
            pl.BlockSpec((KB, NC * H0), lambda k: (k, 0)),
        ],
        out_specs=pl.BlockSpec((m, NC * H0), lambda k: (0, 0)),
        out_shape=jax.ShapeDtypeStruct((m, NC * H0), F32),
        compiler_params=pltpu.CompilerParams(
            dimension_semantics=("arbitrary",)),
    )(x, w_flat)


def _conv_kernel(sup_ref, tmp_ref, side_ref, w1_ref, b1_ref, w2_ref, wb_ref,
                 out_ref, acc_ref, *, emit_basis):
    c = pl.program_id(1)

    @pl.when(c == 0)
    def _():
        acc_ref[...] = jnp.zeros_like(acc_ref)

    acc_ref[...] += jnp.dot(sup_ref[0], tmp_ref[0], preferred_element_type=F32)

    @pl.when(c == NC - 1)
    def _():
        z = jnp.maximum(acc_ref[...], 0.0)
        f = jnp.maximum(
            jnp.dot(side_ref[...], w1_ref[...], preferred_element_type=F32)
            + b1_ref[...], 0.0)
        h = (jnp.dot(z, w2_ref[:H0, :], preferred_element_type=F32)
             + jnp.dot(f, w2_ref[H0:, :], preferred_element_type=F32))
        if emit_basis:
            out_ref[...] = jnp.dot(h, wb_ref[...], preferred_element_type=F32)
        else:
            out_ref[...] = h


def _conv_side(sup, tmp3, side, w1, b1, w2, wb, emit_basis):
    m = sup.shape[1]
    n = sup.shape[2]
    mb = 256
    out_w = NBASIS * H1 if emit_basis else H1
    return pl.pallas_call(
        functools.partial(_conv_kernel, emit_basis=emit_basis),
        grid=(m // mb, NC),
        in_specs=[
            pl.BlockSpec((1, mb, n), lambda i, c: (c, i, 0)),
            pl.BlockSpec((1, n, H0), lambda i, c: (c, 0, 0)),
            pl.BlockSpec((mb, 64), lambda i, c: (i, 0)),
            pl.BlockSpec((64, ID), lambda i, c: (0, 0)),
            pl.BlockSpec((1, ID), lambda i, c: (0, 0)),
            pl.BlockSpec((H0 + ID, H1), lambda i, c: (0, 0)),
            pl.BlockSpec((H1, NBASIS * H1), lambda i, c: (0, 0)),
        ],
        out_specs=pl.BlockSpec((mb, out_w), lambda i, c: (i, 0)),
        out_shape=jax.ShapeDtypeStruct((m, out_w), F32),
        scratch_shapes=[pltpu.VMEM((mb, H0), F32)],
        compiler_params=pltpu.CompilerParams(
            dimension_semantics=("arbitrary", "arbitrary")),
    )(sup, tmp3, side, w1, b1, w2, wb)


SLICE = E_PAD // 16
ECH = 1568
NCHUNK = SLICE // ECH


def _sc_decoder_body(uidx_hbm, vidx_hbm, uh_hbm, vb_hbm, bout_hbm,
                     ut_v, vt_v, uidx_v, vidx_v, out_v):
    wid = lax.axis_index("s") * 2 + lax.axis_index("c")
    b = wid % 2
    k = wid // 2
    pltpu.sync_copy(uh_hbm, ut_v)
    pltpu.sync_copy(vb_hbm.at[pl.ds(b * (NI * H1), NI * H1)], vt_v)

    for ch in range(NCHUNK):
        off = k * SLICE + ch * ECH
        pltpu.sync_copy(uidx_hbm.at[pl.ds(off, ECH)], uidx_v)
        pltpu.sync_copy(vidx_hbm.at[pl.ds(off, ECH)], vidx_v)

        def grp(g, c2):
            e0 = g * 16
            bu = uidx_v[pl.ds(e0, 16)] * H1
            bv = vidx_v[pl.ds(e0, 16)] * H1
            acc = jnp.zeros((16,), F32)
            for f in range(H1):
                gu = plsc.load_gather(ut_v, [bu + f])
                gv = plsc.load_gather(vt_v, [bv + f])
                acc = acc + gu * gv
            out_v[pl.ds(e0, 16)] = acc
            return c2

        lax.fori_loop(0, ECH // 16, grp, 0)
        pltpu.sync_copy(out_v, bout_hbm.at[pl.ds(b * E_PAD + off, ECH)])


@functools.cache
def _sc_decoder_fn():
    return pl.kernel(
        _sc_decoder_body,
        out_type=jax.ShapeDtypeStruct((2 * E_PAD,), F32),
        mesh=plsc.VectorSubcoreMesh(core_axis_name="c", subcore_axis_name="s"),
        compiler_params=pltpu.CompilerParams(needs_layout_passes=False),
        scratch_types=[
            pltpu.VMEM((NU * H1,), F32),
            pltpu.VMEM((NI * H1,), F32),
            pltpu.VMEM((ECH,), jnp.int32),
            pltpu.VMEM((ECH,), jnp.int32),
            pltpu.VMEM((ECH,), F32),
        ],
    )


def _sc_decoder(up, vp, uh_flat, vb_flat):
    bcat = _sc_decoder_fn()(up, vp, uh_flat, vb_flat)
    return bcat[:E_PAD], bcat[E_PAD:]


def _head_kernel(b0_ref, b1_ref, r_ref, amix_ref, mh_ref, loss_ref, rmse_ref):
    b0 = b0_ref[...]
    b1 = b1_ref[...]
    r = r_ref[...]
    o = [b0 * amix_ref[0, k] + b1 * amix_ref[0, NC + k] for k in range(NC)]
    mx = o[0]
    for k in range(1, NC):
        mx = jnp.maximum(mx, o[k])
    e = [jnp.exp(o[k] - mx) for k in range(NC)]
    s = e[0]
    num = e[0]
    for k in range(1, NC):
        s = s + e[k]
        num = num + e[k] * (k + 1.0)
    mh = num / s
    mh_ref[...] = mh

    rows = b0.shape[0]
    cols = b0.shape[1]
    eid = (lax.broadcasted_iota(jnp.int32, (rows, cols), 0) * cols
           + lax.broadcasted_iota(jnp.int32, (rows, cols), 1))
    valid = eid < NE
    o_r = jnp.zeros_like(b0)
    for k in range(NC):
        o_r = o_r + jnp.where(r == k, o[k], 0.0)
    logp_r = o_r - mx - jnp.log(s)
    loss = -jnp.sum(jnp.where(valid, logp_r, 0.0)) / NE
    sq = jnp.where(valid, (mh - (r.astype(F32) + 1.0)) ** 2, 0.0)
    rmse = jnp.sqrt(jnp.sum(sq) / NE)
    loss_ref[0, 0] = loss
    rmse_ref[0, 0] = rmse


def _head(b0, b1, r2d, amix):
    rows, cols = b0.shape
    return pl.pallas_call(
        _head_kernel,
        in_specs=[
            pl.BlockSpec((rows, cols), lambda: (0, 0)),
            pl.BlockSpec((rows, cols), lambda: (0, 0)),
            pl.BlockSpec((rows, cols), lambda: (0, 0)),
            pl.BlockSpec((1, 2 * NC), lambda: (0, 0)),
        ],
        out_specs=[
            pl.BlockSpec((rows, cols), lambda: (0, 0)),
            pl.BlockSpec(memory_space=pltpu.SMEM),
            pl.BlockSpec(memory_space=pltpu.SMEM),
        ],
        out_shape=[
            jax.ShapeDtypeStruct((rows, cols), F32),
            jax.ShapeDtypeStruct((1, 1), F32),
            jax.ShapeDtypeStruct((1, 1), F32),
        ],
    )(b0, b1, r2d, amix)


def kernel(u, v, r, support, support_t, u_side, v_side, u_features,
           v_features, W_gc, W_u1, b_u1, W_v1, b_v1, W_u2, W_v2, W_bil,
           a_mix):
    w_flat = W_gc.transpose(1, 0, 2).reshape(FD, NC * H0)

    tmp_u = _feat_matmul(u_features, w_flat)
    tmp_v = _feat_matmul(v_features, w_flat)
    tmp_u3 = tmp_u.reshape(NU, NC, H0).transpose(1, 0, 2)
    tmp_v3 = tmp_v.reshape(NI, NC, H0).transpose(1, 0, 2)

    wbT = jnp.concatenate([W_bil[0].T, W_bil[1].T], axis=1)
    u_h = _conv_side(support, tmp_v3, u_side, W_u1, b_u1.reshape(1, ID),
                     W_u2, wbT, emit_basis=False)
    vb = _conv_side(support_t, tmp_u3, v_side, W_v1, b_v1.reshape(1, ID),
                    W_v2, wbT, emit_basis=True)
    uh_flat = u_h.reshape(NU * H1)
    vb_flat = vb.reshape(NI, 2, H1).transpose(1, 0, 2).reshape(2 * NI * H1)

    pad = E_PAD - NE
    up = jnp.pad(u.astype(jnp.int32), (0, pad))
    vp = jnp.pad(v.astype(jnp.int32), (0, pad))
    rp = jnp.pad(r.astype(jnp.int32), (0, pad))

    b0, b1 = _sc_decoder(up, vp, uh_flat, vb_flat)

    mh2, loss11, rmse11 = _head(b0.reshape(784, 128), b1.reshape(784, 128),
                                rp.reshape(784, 128), a_mix.reshape(1, 2 * NC))
    m_hat = mh2.reshape(E_PAD)[:NE]
    return (m_hat, loss11[0, 0], rmse11[0, 0])


# speedup vs baseline: 2.8588x; 1.8439x over previous
import functools

import jax
import jax.numpy as jnp
from jax import lax
from jax.experimental import pallas as pl
from jax.experimental.pallas import tpu as pltpu
from jax.experimental.pallas import tpu_sc as plsc

F32 = jnp.float32

NU = 2048
NI = 1536
NC = 5
FD = NU + NI
H0 = 64
H1 = 32
ID = 10
NBASIS = 2
NE = 100000

E_PAD = 100352
KB = 512


def _feat_kernel(x_ref, w_ref, out_ref):
    k = pl.program_id(0)

    @pl.when(k == 0)
    def _():
        out_ref[...] = jnp.zeros_like(out_ref)

    out_ref[...] += jnp.dot(x_ref[...], w_ref[...], preferred_element_type=F32)

    @pl.when(k == pl.num_programs(0) - 1)
    def _():
        for c in range(1, NC):
            out_ref[:, c * H0:(c + 1) * H0] += out_ref[:, (c - 1) * H0:c * H0]


def _feat_matmul(x, w_flat):
    m = x.shape[0]
    return pl.pallas_call(
        _feat_kernel,
        grid=(FD // KB,),
        in_specs=[
            pl.BlockSpec((m, KB), lambda k: (0, k)),
            pl.BlockSpec((KB, NC * H0), lambda k: (k, 0)),
        ],
        out_specs=pl.BlockSpec((m, NC * H0), lambda k: (0, 0)),
        out_shape=jax.ShapeDtypeStruct((m, NC * H0), F32),
        compiler_params=pltpu.CompilerParams(
            dimension_semantics=("arbitrary",)),
    )(x, w_flat)


def _conv_kernel(sup_ref, tmp_ref, side_ref, w1_ref, b1_ref, w2_ref, wb_ref,
                 out_ref, acc_ref, *, emit_basis):
    c = pl.program_id(1)

    @pl.when(c == 0)
    def _():
        acc_ref[...] = jnp.zeros_like(acc_ref)

    acc_ref[...] += jnp.dot(sup_ref[0], tmp_ref[0], preferred_element_type=F32)

    @pl.when(c == NC - 1)
    def _():
        z = jnp.maximum(acc_ref[...], 0.0)
        f = jnp.maximum(
            jnp.dot(side_ref[...], w1_ref[...], preferred_element_type=F32)
            + b1_ref[...], 0.0)
        h = (jnp.dot(z, w2_ref[:H0, :], preferred_element_type=F32)
             + jnp.dot(f, w2_ref[H0:, :], preferred_element_type=F32))
        if emit_basis:
            out_ref[...] = jnp.dot(h, wb_ref[...], preferred_element_type=F32)
        else:
            out_ref[...] = h


def _conv_side(sup, tmp3, side, w1, b1, w2, wb, emit_basis):
    m = sup.shape[1]
    n = sup.shape[2]
    mb = 256
    out_w = NBASIS * H1 if emit_basis else H1
    return pl.pallas_call(
        functools.partial(_conv_kernel, emit_basis=emit_basis),
        grid=(m // mb, NC),
        in_specs=[
            pl.BlockSpec((1, mb, n), lambda i, c: (c, i, 0)),
            pl.BlockSpec((1, n, H0), lambda i, c: (c, 0, 0)),
            pl.BlockSpec((mb, 64), lambda i, c: (i, 0)),
            pl.BlockSpec((64, ID), lambda i, c: (0, 0)),
            pl.BlockSpec((1, ID), lambda i, c: (0, 0)),
            pl.BlockSpec((H0 + ID, H1), lambda i, c: (0, 0)),
            pl.BlockSpec((H1, NBASIS * H1), lambda i, c: (0, 0)),
        ],
        out_specs=pl.BlockSpec((mb, out_w), lambda i, c: (i, 0)),
        out_shape=jax.ShapeDtypeStruct((m, out_w), F32),
        scratch_shapes=[pltpu.VMEM((mb, H0), F32)],
        compiler_params=pltpu.CompilerParams(
            dimension_semantics=("arbitrary", "arbitrary")),
    )(sup, tmp3, side, w1, b1, w2, wb)


SLICE = E_PAD // 32
ECH = 1568
NCHUNK = SLICE // ECH


def _sc_decoder_body(uidx_hbm, vidx_hbm, uh_hbm, vb_hbm, bout_hbm,
                     ut_v, vt_v, uidx_v, vidx_v, b0_v, b1_v):
    wid = lax.axis_index("s") * 2 + lax.axis_index("c")
    pltpu.sync_copy(uh_hbm, ut_v)
    pltpu.sync_copy(vb_hbm, vt_v)

    for ch in range(NCHUNK):
        off = wid * SLICE + ch * ECH
        pltpu.sync_copy(uidx_hbm.at[pl.ds(off, ECH)], uidx_v)
        pltpu.sync_copy(vidx_hbm.at[pl.ds(off, ECH)], vidx_v)

        def grp(g, c2):
            e0 = g * 16
            iu = uidx_v[pl.ds(e0, 16)]
            iv = vidx_v[pl.ds(e0, 16)]
            acc0 = jnp.zeros((16,), F32)
            acc1 = jnp.zeros((16,), F32)
            for f in range(H1):
                gu = plsc.load_gather(ut_v, [iu + f * NU])
                gp = plsc.load_gather(vt_v, [iv + f * NI])
                v0, v1 = plsc.unpack(plsc.bitcast(gp, jnp.bfloat16),
                                     format=plsc.PackFormat.INTERLEAVED)
                acc0 = acc0 + gu * v0
                acc1 = acc1 + gu * v1
            b0_v[pl.ds(e0, 16)] = acc0
            b1_v[pl.ds(e0, 16)] = acc1
            return c2

        lax.fori_loop(0, ECH // 16, grp, 0)
        pltpu.sync_copy(b0_v, bout_hbm.at[pl.ds(off, ECH)])
        pltpu.sync_copy(b1_v, bout_hbm.at[pl.ds(E_PAD + off, ECH)])


@functools.cache
def _sc_decoder_fn():
    return pl.kernel(
        _sc_decoder_body,
        out_type=jax.ShapeDtypeStruct((2 * E_PAD,), F32),
        mesh=plsc.VectorSubcoreMesh(core_axis_name="c", subcore_axis_name="s"),
        compiler_params=pltpu.CompilerParams(needs_layout_passes=False),
        scratch_types=[
            pltpu.VMEM((NU * H1,), F32),
            pltpu.VMEM((NI * H1,), jnp.int32),
            pltpu.VMEM((ECH,), jnp.int32),
            pltpu.VMEM((ECH,), jnp.int32),
            pltpu.VMEM((ECH,), F32),
            pltpu.VMEM((ECH,), F32),
        ],
    )


def _sc_decoder(up, vp, uht_flat, vbp_flat):
    bcat = _sc_decoder_fn()(up, vp, uht_flat, vbp_flat)
    return bcat[:E_PAD], bcat[E_PAD:]


def _head_kernel(b0_ref, b1_ref, r_ref, amix_ref, mh_ref, loss_ref, rmse_ref):
    b0 = b0_ref[...]
    b1 = b1_ref[...]
    r = r_ref[...]
    o = [b0 * amix_ref[0, k] + b1 * amix_ref[0, NC + k] for k in range(NC)]
    mx = o[0]
    for k in range(1, NC):
        mx = jnp.maximum(mx, o[k])
    e = [jnp.exp(o[k] - mx) for k in range(NC)]
    s = e[0]
    num = e[0]
    for k in range(1, NC):
        s = s + e[k]
        num = num + e[k] * (k + 1.0)
    mh = num / s
    mh_ref[...] = mh

    rows = b0.shape[0]
    cols = b0.shape[1]
    eid = (lax.broadcasted_iota(jnp.int32, (rows, cols), 0) * cols
           + lax.broadcasted_iota(jnp.int32, (rows, cols), 1))
    valid = eid < NE
    o_r = jnp.zeros_like(b0)
    for k in range(NC):
        o_r = o_r + jnp.where(r == k, o[k], 0.0)
    logp_r = o_r - mx - jnp.log(s)
    loss = -jnp.sum(jnp.where(valid, logp_r, 0.0)) / NE
    sq = jnp.where(valid, (mh - (r.astype(F32) + 1.0)) ** 2, 0.0)
    rmse = jnp.sqrt(jnp.sum(sq) / NE)
    loss_ref[0, 0] = loss
    rmse_ref[0, 0] = rmse


def _head(b0, b1, r2d, amix):
    rows, cols = b0.shape
    return pl.pallas_call(
        _head_kernel,
        in_specs=[
            pl.BlockSpec((rows, cols), lambda: (0, 0)),
            pl.BlockSpec((rows, cols), lambda: (0, 0)),
            pl.BlockSpec((rows, cols), lambda: (0, 0)),
            pl.BlockSpec((1, 2 * NC), lambda: (0, 0)),
        ],
        out_specs=[
            pl.BlockSpec((rows, cols), lambda: (0, 0)),
            pl.BlockSpec(memory_space=pltpu.SMEM),
            pl.BlockSpec(memory_space=pltpu.SMEM),
        ],
        out_shape=[
            jax.ShapeDtypeStruct((rows, cols), F32),
            jax.ShapeDtypeStruct((1, 1), F32),
            jax.ShapeDtypeStruct((1, 1), F32),
        ],
    )(b0, b1, r2d, amix)


def kernel(u, v, r, support, support_t, u_side, v_side, u_features,
           v_features, W_gc, W_u1, b_u1, W_v1, b_v1, W_u2, W_v2, W_bil,
           a_mix):
    w_flat = W_gc.transpose(1, 0, 2).reshape(FD, NC * H0)

    tmp_u = _feat_matmul(u_features, w_flat)
    tmp_v = _feat_matmul(v_features, w_flat)
    tmp_u3 = tmp_u.reshape(NU, NC, H0).transpose(1, 0, 2)
    tmp_v3 = tmp_v.reshape(NI, NC, H0).transpose(1, 0, 2)

    wbT = jnp.concatenate([W_bil[0].T, W_bil[1].T], axis=1)
    u_h = _conv_side(support, tmp_v3, u_side, W_u1, b_u1.reshape(1, ID),
                     W_u2, wbT, emit_basis=False)
    vb = _conv_side(support_t, tmp_u3, v_side, W_v1, b_v1.reshape(1, ID),
                    W_v2, wbT, emit_basis=True)
    uht_flat = u_h.T.reshape(H1 * NU)
    vb3 = jnp.stack([vb[:, :H1].T, vb[:, H1:].T], axis=-1)
    vbp_flat = lax.bitcast_convert_type(
        vb3.astype(jnp.bfloat16), jnp.int32).reshape(H1 * NI)

    pad = E_PAD - NE
    up = jnp.pad(u.astype(jnp.int32), (0, pad))
    vp = jnp.pad(v.astype(jnp.int32), (0, pad))
    rp = jnp.pad(r.astype(jnp.int32), (0, pad))

    b0, b1 = _sc_decoder(up, vp, uht_flat, vbp_flat)

    mh2, loss11, rmse11 = _head(b0.reshape(784, 128), b1.reshape(784, 128),
                                rp.reshape(784, 128), a_mix.reshape(1, 2 * NC))
    m_hat = mh2.reshape(E_PAD)[:NE]
    return (m_hat, loss11[0, 0], rmse11[0, 0])


# speedup vs baseline: 2.8697x; 1.0038x over previous
import functools

import jax
import jax.numpy as jnp
from jax import lax
from jax.experimental import pallas as pl
from jax.experimental.pallas import tpu as pltpu
from jax.experimental.pallas import tpu_sc as plsc

F32 = jnp.float32

NU = 2048
NI = 1536
NC = 5
FD = NU + NI
H0 = 64
H1 = 32
ID = 10
NBASIS = 2
NE = 100000

E_PAD = 100352
KB = 512


def _feat_kernel(x_ref, w_ref, out_ref):
    k = pl.program_id(0)

    @pl.when(k == 0)
    def _():
        out_ref[...] = jnp.zeros_like(out_ref)

    out_ref[...] += jnp.dot(x_ref[...].astype(jnp.bfloat16),
                            w_ref[...].astype(jnp.bfloat16),
                            preferred_element_type=F32)

    @pl.when(k == pl.num_programs(0) - 1)
    def _():
        for c in range(1, NC):
            out_ref[:, c * H0:(c + 1) * H0] += out_ref[:, (c - 1) * H0:c * H0]


def _feat_matmul(x, w_flat):
    m = x.shape[0]
    return pl.pallas_call(
        _feat_kernel,
        grid=(FD // KB,),
        in_specs=[
            pl.BlockSpec((m, KB), lambda k: (0, k)),
            pl.BlockSpec((KB, NC * H0), lambda k: (k, 0)),
        ],
        out_specs=pl.BlockSpec((m, NC * H0), lambda k: (0, 0)),
        out_shape=jax.ShapeDtypeStruct((m, NC * H0), F32),
        compiler_params=pltpu.CompilerParams(
            dimension_semantics=("arbitrary",)),
    )(x, w_flat)


def _conv_kernel(sup_ref, tmp_ref, side_ref, w1_ref, b1_ref, w2_ref, wb_ref,
                 out_ref, acc_ref, *, emit_basis):
    c = pl.program_id(1)

    @pl.when(c == 0)
    def _():
        acc_ref[...] = jnp.zeros_like(acc_ref)

    acc_ref[...] += jnp.dot(sup_ref[0].astype(jnp.bfloat16),
                            tmp_ref[0].astype(jnp.bfloat16),
                            preferred_element_type=F32)

    @pl.when(c == NC - 1)
    def _():
        z = jnp.maximum(acc_ref[...], 0.0)
        f = jnp.maximum(
            jnp.dot(side_ref[...], w1_ref[...], preferred_element_type=F32)
            + b1_ref[...], 0.0)
        h = (jnp.dot(z, w2_ref[:H0, :], preferred_element_type=F32)
             + jnp.dot(f, w2_ref[H0:, :], preferred_element_type=F32))
        if emit_basis:
            out_ref[...] = jnp.dot(h, wb_ref[...], preferred_element_type=F32)
        else:
            out_ref[...] = h


def _conv_side(sup, tmp3, side, w1, b1, w2, wb, emit_basis):
    m = sup.shape[1]
    n = sup.shape[2]
    mb = 256
    out_w = NBASIS * H1 if emit_basis else H1
    return pl.pallas_call(
        functools.partial(_conv_kernel, emit_basis=emit_basis),
        grid=(m // mb, NC),
        in_specs=[
            pl.BlockSpec((1, mb, n), lambda i, c: (c, i, 0)),
            pl.BlockSpec((1, n, H0), lambda i, c: (c, 0, 0)),
            pl.BlockSpec((mb, 64), lambda i, c: (i, 0)),
            pl.BlockSpec((64, ID), lambda i, c: (0, 0)),
            pl.BlockSpec((1, ID), lambda i, c: (0, 0)),
            pl.BlockSpec((H0 + ID, H1), lambda i, c: (0, 0)),
            pl.BlockSpec((H1, NBASIS * H1), lambda i, c: (0, 0)),
        ],
        out_specs=pl.BlockSpec((mb, out_w), lambda i, c: (i, 0)),
        out_shape=jax.ShapeDtypeStruct((m, out_w), F32),
        scratch_shapes=[pltpu.VMEM((mb, H0), F32)],
        compiler_params=pltpu.CompilerParams(
            dimension_semantics=("arbitrary", "arbitrary")),
    )(sup, tmp3, side, w1, b1, w2, wb)


SLICE = E_PAD // 32
ECH = 1568
NCHUNK = SLICE // ECH


def _sc_decoder_body(uidx_hbm, vidx_hbm, uh_hbm, vb_hbm, bout_hbm,
                     ut_v, vt_v, uidx_v, vidx_v, b0_v, b1_v):
    wid = lax.axis_index("s") * 2 + lax.axis_index("c")
    pltpu.sync_copy(uh_hbm, ut_v)
    pltpu.sync_copy(vb_hbm, vt_v)

    for ch in range(NCHUNK):
        off = wid * SLICE + ch * ECH
        pltpu.sync_copy(uidx_hbm.at[pl.ds(off, ECH)], uidx_v)
        pltpu.sync_copy(vidx_hbm.at[pl.ds(off, ECH)], vidx_v)

        def grp(g, c2):
            e0 = g * 16
            iu = uidx_v[pl.ds(e0, 16)]
            iv = vidx_v[pl.ds(e0, 16)]
            acc0 = jnp.zeros((16,), F32)
            acc1 = jnp.zeros((16,), F32)
            for f in range(H1):
                gu = plsc.load_gather(ut_v, [iu + f * NU])
                gp = plsc.load_gather(vt_v, [iv + f * NI])
                v0, v1 = plsc.unpack(plsc.bitcast(gp, jnp.bfloat16),
                                     format=plsc.PackFormat.INTERLEAVED)
                acc0 = acc0 + gu * v0
                acc1 = acc1 + gu * v1
            b0_v[pl.ds(e0, 16)] = acc0
            b1_v[pl.ds(e0, 16)] = acc1
            return c2

        lax.fori_loop(0, ECH // 16, grp, 0)
        pltpu.sync_copy(b0_v, bout_hbm.at[pl.ds(off, ECH)])
        pltpu.sync_copy(b1_v, bout_hbm.at[pl.ds(E_PAD + off, ECH)])


@functools.cache
def _sc_decoder_fn():
    return pl.kernel(
        _sc_decoder_body,
        out_type=jax.ShapeDtypeStruct((2 * E_PAD,), F32),
        mesh=plsc.VectorSubcoreMesh(core_axis_name="c", subcore_axis_name="s"),
        compiler_params=pltpu.CompilerParams(needs_layout_passes=False),
        scratch_types=[
            pltpu.VMEM((NU * H1,), F32),
            pltpu.VMEM((NI * H1,), jnp.int32),
            pltpu.VMEM((ECH,), jnp.int32),
            pltpu.VMEM((ECH,), jnp.int32),
            pltpu.VMEM((ECH,), F32),
            pltpu.VMEM((ECH,), F32),
        ],
    )


def _sc_decoder(up, vp, uht_flat, vbp_flat):
    bcat = _sc_decoder_fn()(up, vp, uht_flat, vbp_flat)
    return bcat[:E_PAD], bcat[E_PAD:]


def _head_kernel(b0_ref, b1_ref, r_ref, amix_ref, mh_ref, loss_ref, rmse_ref):
    b0 = b0_ref[...]
    b1 = b1_ref[...]
    r = r_ref[...]
    o = [b0 * amix_ref[0, k] + b1 * amix_ref[0, NC + k] for k in range(NC)]
    mx = o[0]
    for k in range(1, NC):
        mx = jnp.maximum(mx, o[k])
    e = [jnp.exp(o[k] - mx) for k in range(NC)]
    s = e[0]
    num = e[0]
    for k in range(1, NC):
        s = s + e[k]
        num = num + e[k] * (k + 1.0)
    mh = num / s
    mh_ref[...] = mh

    rows = b0.shape[0]
    cols = b0.shape[1]
    eid = (lax.broadcasted_iota(jnp.int32, (rows, cols), 0) * cols
           + lax.broadcasted_iota(jnp.int32, (rows, cols), 1))
    valid = eid < NE
    o_r = jnp.zeros_like(b0)
    for k in range(NC):
        o_r = o_r + jnp.where(r == k, o[k], 0.0)
    logp_r = o_r - mx - jnp.log(s)
    loss = -jnp.sum(jnp.where(valid, logp_r, 0.0)) / NE
    sq = jnp.where(valid, (mh - (r.astype(F32) + 1.0)) ** 2, 0.0)
    rmse = jnp.sqrt(jnp.sum(sq) / NE)
    loss_ref[0, 0] = loss
    rmse_ref[0, 0] = rmse


def _head(b0, b1, r2d, amix):
    rows, cols = b0.shape
    return pl.pallas_call(
        _head_kernel,
        in_specs=[
            pl.BlockSpec((rows, cols), lambda: (0, 0)),
            pl.BlockSpec((rows, cols), lambda: (0, 0)),
            pl.BlockSpec((rows, cols), lambda: (0, 0)),
            pl.BlockSpec((1, 2 * NC), lambda: (0, 0)),
        ],
        out_specs=[
            pl.BlockSpec((rows, cols), lambda: (0, 0)),
            pl.BlockSpec(memory_space=pltpu.SMEM),
            pl.BlockSpec(memory_space=pltpu.SMEM),
        ],
        out_shape=[
            jax.ShapeDtypeStruct((rows, cols), F32),
            jax.ShapeDtypeStruct((1, 1), F32),
            jax.ShapeDtypeStruct((1, 1), F32),
        ],
    )(b0, b1, r2d, amix)


def kernel(u, v, r, support, support_t, u_side, v_side, u_features,
           v_features, W_gc, W_u1, b_u1, W_v1, b_v1, W_u2, W_v2, W_bil,
           a_mix):
    w_flat = W_gc.transpose(1, 0, 2).reshape(FD, NC * H0)

    tmp_u = _feat_matmul(u_features, w_flat)
    tmp_v = _feat_matmul(v_features, w_flat)
    tmp_u3 = tmp_u.reshape(NU, NC, H0).transpose(1, 0, 2)
    tmp_v3 = tmp_v.reshape(NI, NC, H0).transpose(1, 0, 2)

    wbT = jnp.concatenate([W_bil[0].T, W_bil[1].T], axis=1)
    u_h = _conv_side(support, tmp_v3, u_side, W_u1, b_u1.reshape(1, ID),
                     W_u2, wbT, emit_basis=False)
    vb = _conv_side(support_t, tmp_u3, v_side, W_v1, b_v1.reshape(1, ID),
                    W_v2, wbT, emit_basis=True)
    uht_flat = u_h.T.reshape(H1 * NU)
    vb3 = jnp.stack([vb[:, :H1].T, vb[:, H1:].T], axis=-1)
    vbp_flat = lax.bitcast_convert_type(
        vb3.astype(jnp.bfloat16), jnp.int32).reshape(H1 * NI)

    pad = E_PAD - NE
    up = jnp.pad(u.astype(jnp.int32), (0, pad))
    vp = jnp.pad(v.astype(jnp.int32), (0, pad))
    rp = jnp.pad(r.astype(jnp.int32), (0, pad))

    b0, b1 = _sc_decoder(up, vp, uht_flat, vbp_flat)

    mh2, loss11, rmse11 = _head(b0.reshape(784, 128), b1.reshape(784, 128),
                                rp.reshape(784, 128), a_mix.reshape(1, 2 * NC))
    m_hat = mh2.reshape(E_PAD)[:NE]
    return (m_hat, loss11[0, 0], rmse11[0, 0])


# speedup vs baseline: 3.6509x; 1.2722x over previous
import functools

import jax
import jax.numpy as jnp
from jax import lax
from jax.experimental import pallas as pl
from jax.experimental.pallas import tpu as pltpu
from jax.experimental.pallas import tpu_sc as plsc

F32 = jnp.float32

NU = 2048
NI = 1536
NC = 5
FD = NU + NI
H0 = 64
H1 = 32
ID = 10
NBASIS = 2
NE = 100000

E_PAD = 100352
KB = 512


def _feat_kernel(x_ref, w_ref, out_ref):
    k = pl.program_id(0)

    @pl.when(k == 0)
    def _():
        out_ref[...] = jnp.zeros_like(out_ref)

    wcat = jnp.concatenate([w_ref[c] for c in range(NC)], axis=1)
    out_ref[...] += jnp.dot(x_ref[...].astype(jnp.bfloat16),
                            wcat.astype(jnp.bfloat16),
                            preferred_element_type=F32)

    @pl.when(k == pl.num_programs(0) - 1)
    def _():
        for c in range(1, NC):
            out_ref[:, c * H0:(c + 1) * H0] += out_ref[:, (c - 1) * H0:c * H0]


def _feat_matmul(x, w_flat):
    m = x.shape[0]
    return pl.pallas_call(
        _feat_kernel,
        grid=(FD // KB,),
        in_specs=[
            pl.BlockSpec((m, KB), lambda k: (0, k)),
            pl.BlockSpec((NC, KB, H0), lambda k: (0, k, 0)),
        ],
        out_specs=pl.BlockSpec((m, NC * H0), lambda k: (0, 0)),
        out_shape=jax.ShapeDtypeStruct((m, NC * H0), F32),
        compiler_params=pltpu.CompilerParams(
            dimension_semantics=("arbitrary",)),
    )(x, w_flat)


def _conv_kernel(sup_ref, tmp_ref, side_ref, w1_ref, b1_ref, w2_ref, wb_ref,
                 out_ref, *, emit_basis):
    acc = None
    for c in range(NC):
        part = jnp.dot(sup_ref[c].astype(jnp.bfloat16),
                       tmp_ref[:, c * H0:(c + 1) * H0].astype(jnp.bfloat16),
                       preferred_element_type=F32)
        acc = part if acc is None else acc + part
    z = jnp.maximum(acc, 0.0)
    f = jnp.maximum(
        jnp.dot(side_ref[...], w1_ref[...], preferred_element_type=F32)
        + b1_ref[...], 0.0)
    h = (jnp.dot(z, w2_ref[:H0, :], preferred_element_type=F32)
         + jnp.dot(f, w2_ref[H0:, :], preferred_element_type=F32))
    if emit_basis:
        out_ref[...] = jnp.dot(h, wb_ref[...], preferred_element_type=F32)
    else:
        out_ref[...] = h


def _conv_side(sup, tmp, side, w1, b1, w2, wb, emit_basis):
    m = sup.shape[1]
    n = sup.shape[2]
    mb = 256
    out_w = NBASIS * H1 if emit_basis else H1
    return pl.pallas_call(
        functools.partial(_conv_kernel, emit_basis=emit_basis),
        grid=(m // mb,),
        in_specs=[
            pl.BlockSpec((NC, mb, n), lambda i: (0, i, 0)),
            pl.BlockSpec((n, NC * H0), lambda i: (0, 0)),
            pl.BlockSpec((mb, 64), lambda i: (i, 0)),
            pl.BlockSpec((64, ID), lambda i: (0, 0)),
            pl.BlockSpec((1, ID), lambda i: (0, 0)),
            pl.BlockSpec((H0 + ID, H1), lambda i: (0, 0)),
            pl.BlockSpec((H1, NBASIS * H1), lambda i: (0, 0)),
        ],
        out_specs=pl.BlockSpec((mb, out_w), lambda i: (i, 0)),
        out_shape=jax.ShapeDtypeStruct((m, out_w), F32),
        compiler_params=pltpu.CompilerParams(
            dimension_semantics=("arbitrary",)),
    )(sup, tmp, side, w1, b1, w2, wb)


SLICE = E_PAD // 32
ECH = 1568
NCHUNK = SLICE // ECH


def _sc_decoder_body(uidx_hbm, vidx_hbm, uh_hbm, vb_hbm, bout_hbm,
                     ut_v, vt_v, uidx_v, vidx_v, b0_v, b1_v):
    wid = lax.axis_index("s") * 2 + lax.axis_index("c")
    pltpu.sync_copy(uh_hbm, ut_v)
    pltpu.sync_copy(vb_hbm, vt_v)

    for ch in range(NCHUNK):
        off = wid * SLICE + ch * ECH
        pltpu.sync_copy(uidx_hbm.at[pl.ds(off, ECH)], uidx_v)
        pltpu.sync_copy(vidx_hbm.at[pl.ds(off, ECH)], vidx_v)

        def grp(g, c2):
            e0 = g * 16
            iu = uidx_v[pl.ds(e0, 16)]
            iv = vidx_v[pl.ds(e0, 16)]
            acc0 = jnp.zeros((16,), F32)
            acc1 = jnp.zeros((16,), F32)
            for f in range(H1):
                gu = plsc.load_gather(ut_v, [iu + f * NU])
                gp = plsc.load_gather(vt_v, [iv + f * NI])
                v0, v1 = plsc.unpack(plsc.bitcast(gp, jnp.bfloat16),
                                     format=plsc.PackFormat.INTERLEAVED)
                acc0 = acc0 + gu * v0
                acc1 = acc1 + gu * v1
            b0_v[pl.ds(e0, 16)] = acc0
            b1_v[pl.ds(e0, 16)] = acc1
            return c2

        lax.fori_loop(0, ECH // 16, grp, 0)
        pltpu.sync_copy(b0_v, bout_hbm.at[pl.ds(off, ECH)])
        pltpu.sync_copy(b1_v, bout_hbm.at[pl.ds(E_PAD + off, ECH)])


@functools.cache
def _sc_decoder_fn():
    return pl.kernel(
        _sc_decoder_body,
        out_type=jax.ShapeDtypeStruct((2 * E_PAD,), F32),
        mesh=plsc.VectorSubcoreMesh(core_axis_name="c", subcore_axis_name="s"),
        compiler_params=pltpu.CompilerParams(needs_layout_passes=False),
        scratch_types=[
            pltpu.VMEM((NU * H1,), F32),
            pltpu.VMEM((NI * H1,), jnp.int32),
            pltpu.VMEM((ECH,), jnp.int32),
            pltpu.VMEM((ECH,), jnp.int32),
            pltpu.VMEM((ECH,), F32),
            pltpu.VMEM((ECH,), F32),
        ],
    )


def _sc_decoder(up, vp, uht_flat, vbp_flat):
    bcat = _sc_decoder_fn()(up, vp, uht_flat, vbp_flat)
    return bcat[:E_PAD], bcat[E_PAD:]


def _head_kernel(b0_ref, b1_ref, r_ref, amix_ref, mh_ref, loss_ref, rmse_ref):
    b0 = b0_ref[...]
    b1 = b1_ref[...]
    r = r_ref[...]
    o = [b0 * amix_ref[0, k] + b1 * amix_ref[0, NC + k] for k in range(NC)]
    mx = o[0]
    for k in range(1, NC):
        mx = jnp.maximum(mx, o[k])
    e = [jnp.exp(o[k] - mx) for k in range(NC)]
    s = e[0]
    num = e[0]
    for k in range(1, NC):
        s = s + e[k]
        num = num + e[k] * (k + 1.0)
    mh = num / s
    mh_ref[...] = mh

    rows = b0.shape[0]
    cols = b0.shape[1]
    eid = (lax.broadcasted_iota(jnp.int32, (rows, cols), 0) * cols
           + lax.broadcasted_iota(jnp.int32, (rows, cols), 1))
    valid = eid < NE
    o_r = jnp.zeros_like(b0)
    for k in range(NC):
        o_r = o_r + jnp.where(r == k, o[k], 0.0)
    logp_r = o_r - mx - jnp.log(s)
    loss = -jnp.sum(jnp.where(valid, logp_r, 0.0)) / NE
    sq = jnp.where(valid, (mh - (r.astype(F32) + 1.0)) ** 2, 0.0)
    rmse = jnp.sqrt(jnp.sum(sq) / NE)
    loss_ref[0, 0] = loss
    rmse_ref[0, 0] = rmse


def _head(b0, b1, r2d, amix):
    rows, cols = b0.shape
    return pl.pallas_call(
        _head_kernel,
        in_specs=[
            pl.BlockSpec((rows, cols), lambda: (0, 0)),
            pl.BlockSpec((rows, cols), lambda: (0, 0)),
            pl.BlockSpec((rows, cols), lambda: (0, 0)),
            pl.BlockSpec((1, 2 * NC), lambda: (0, 0)),
        ],
        out_specs=[
            pl.BlockSpec((rows, cols), lambda: (0, 0)),
            pl.BlockSpec(memory_space=pltpu.SMEM),
            pl.BlockSpec(memory_space=pltpu.SMEM),
        ],
        out_shape=[
            jax.ShapeDtypeStruct((rows, cols), F32),
            jax.ShapeDtypeStruct((1, 1), F32),
            jax.ShapeDtypeStruct((1, 1), F32),
        ],
    )(b0, b1, r2d, amix)


def kernel(u, v, r, support, support_t, u_side, v_side, u_features,
           v_features, W_gc, W_u1, b_u1, W_v1, b_v1, W_u2, W_v2, W_bil,
           a_mix):
    tmp_u = _feat_matmul(u_features, W_gc)
    tmp_v = _feat_matmul(v_features, W_gc)

    wbT = jnp.concatenate([W_bil[0].T, W_bil[1].T], axis=1)
    u_h = _conv_side(support, tmp_v, u_side, W_u1, b_u1.reshape(1, ID),
                     W_u2, wbT, emit_basis=False)
    vb = _conv_side(support_t, tmp_u, v_side, W_v1, b_v1.reshape(1, ID),
                    W_v2, wbT, emit_basis=True)
    uht_flat = u_h.T.reshape(H1 * NU)
    vb3 = jnp.stack([vb[:, :H1].T, vb[:, H1:].T], axis=-1)
    vbp_flat = lax.bitcast_convert_type(
        vb3.astype(jnp.bfloat16), jnp.int32).reshape(H1 * NI)

    pad = E_PAD - NE
    up = jnp.pad(u.astype(jnp.int32), (0, pad))
    vp = jnp.pad(v.astype(jnp.int32), (0, pad))
    rp = jnp.pad(r.astype(jnp.int32), (0, pad))

    b0, b1 = _sc_decoder(up, vp, uht_flat, vbp_flat)

    mh2, loss11, rmse11 = _head(b0.reshape(784, 128), b1.reshape(784, 128),
                                rp.reshape(784, 128), a_mix.reshape(1, 2 * NC))
    m_hat = mh2.reshape(E_PAD)[:NE]
    return (m_hat, loss11[0, 0], rmse11[0, 0])


# speedup vs baseline: 3.7461x; 1.0261x over previous
import functools

import jax
import jax.numpy as jnp
from jax import lax
from jax.experimental import pallas as pl
from jax.experimental.pallas import tpu as pltpu
from jax.experimental.pallas import tpu_sc as plsc

F32 = jnp.float32

NU = 2048
NI = 1536
NC = 5
FD = NU + NI
H0 = 64
H1 = 32
ID = 10
NBASIS = 2
NE = 100000

E_PAD = 100352
KB = 512


def _feat_kernel(x_ref, w_ref, out_ref):
    k = pl.program_id(0)

    @pl.when(k == 0)
    def _():
        out_ref[...] = jnp.zeros_like(out_ref)

    wcat = jnp.concatenate([w_ref[c] for c in range(NC)], axis=1)
    out_ref[...] += jnp.dot(x_ref[...].astype(jnp.bfloat16),
                            wcat.astype(jnp.bfloat16),
                            preferred_element_type=F32)

    @pl.when(k == pl.num_programs(0) - 1)
    def _():
        for c in range(1, NC):
            out_ref[:, c * H0:(c + 1) * H0] += out_ref[:, (c - 1) * H0:c * H0]


def _feat_matmul(x, w_flat):
    m = x.shape[0]
    return pl.pallas_call(
        _feat_kernel,
        grid=(FD // KB,),
        in_specs=[
            pl.BlockSpec((m, KB), lambda k: (0, k)),
            pl.BlockSpec((NC, KB, H0), lambda k: (0, k, 0)),
        ],
        out_specs=pl.BlockSpec((m, NC * H0), lambda k: (0, 0)),
        out_shape=jax.ShapeDtypeStruct((m, NC * H0), F32),
        compiler_params=pltpu.CompilerParams(
            dimension_semantics=("arbitrary",)),
    )(x, w_flat)


def _conv_kernel(sup_ref, tmp_ref, side_ref, w1_ref, b1_ref, w2_ref, wb_ref,
                 out_ref, *, emit_basis):
    acc = None
    for c in range(NC):
        part = jnp.dot(sup_ref[c].astype(jnp.bfloat16),
                       tmp_ref[:, c * H0:(c + 1) * H0].astype(jnp.bfloat16),
                       preferred_element_type=F32)
        acc = part if acc is None else acc + part
    z = jnp.maximum(acc, 0.0)
    f = jnp.maximum(
        jnp.dot(side_ref[...], w1_ref[...], preferred_element_type=F32)
        + b1_ref[...], 0.0)
    h = (jnp.dot(z, w2_ref[:H0, :], preferred_element_type=F32)
         + jnp.dot(f, w2_ref[H0:, :], preferred_element_type=F32))
    if emit_basis:
        out_ref[...] = jnp.dot(h, wb_ref[...], preferred_element_type=F32)
    else:
        out_ref[...] = h


def _conv_side(sup, tmp, side, w1, b1, w2, wb, emit_basis):
    m = sup.shape[1]
    n = sup.shape[2]
    mb = 256
    out_w = NBASIS * H1 if emit_basis else H1
    return pl.pallas_call(
        functools.partial(_conv_kernel, emit_basis=emit_basis),
        grid=(m // mb,),
        in_specs=[
            pl.BlockSpec((NC, mb, n), lambda i: (0, i, 0)),
            pl.BlockSpec((n, NC * H0), lambda i: (0, 0)),
            pl.BlockSpec((mb, 64), lambda i: (i, 0)),
            pl.BlockSpec((64, ID), lambda i: (0, 0)),
            pl.BlockSpec((1, ID), lambda i: (0, 0)),
            pl.BlockSpec((H0 + ID, H1), lambda i: (0, 0)),
            pl.BlockSpec((H1, NBASIS * H1), lambda i: (0, 0)),
        ],
        out_specs=pl.BlockSpec((mb, out_w), lambda i: (i, 0)),
        out_shape=jax.ShapeDtypeStruct((m, out_w), F32),
        compiler_params=pltpu.CompilerParams(
            dimension_semantics=("arbitrary",)),
    )(sup, tmp, side, w1, b1, w2, wb)


SLICE = E_PAD // 32
ECH = 1568
NCHUNK = SLICE // ECH


def _sc_decoder_body(uidx_hbm, vidx_hbm, uh_hbm, vb_hbm, bout_hbm,
                     ut_v, vt_v, uidx_v, vidx_v, b0_v, b1_v):
    wid = lax.axis_index("s") * 2 + lax.axis_index("c")
    pltpu.sync_copy(uh_hbm, ut_v)
    pltpu.sync_copy(vb_hbm, vt_v)

    for ch in range(NCHUNK):
        off = wid * SLICE + ch * ECH
        pltpu.sync_copy(uidx_hbm.at[pl.ds(off, ECH)], uidx_v)
        pltpu.sync_copy(vidx_hbm.at[pl.ds(off, ECH)], vidx_v)

        def grp(g, c2):
            e0 = g * 16
            iu = uidx_v[pl.ds(e0, 16)]
            iv = vidx_v[pl.ds(e0, 16)]
            acc0 = jnp.zeros((16,), F32)
            acc1 = jnp.zeros((16,), F32)
            for fp in range(H1 // 2):
                gup = plsc.load_gather(ut_v, [iu + fp * NU])
                u0, u1 = plsc.unpack(plsc.bitcast(gup, jnp.bfloat16),
                                     format=plsc.PackFormat.INTERLEAVED)
                gp0 = plsc.load_gather(vt_v, [iv + (2 * fp) * NI])
                v00, v10 = plsc.unpack(plsc.bitcast(gp0, jnp.bfloat16),
                                       format=plsc.PackFormat.INTERLEAVED)
                gp1 = plsc.load_gather(vt_v, [iv + (2 * fp + 1) * NI])
                v01, v11 = plsc.unpack(plsc.bitcast(gp1, jnp.bfloat16),
                                       format=plsc.PackFormat.INTERLEAVED)
                acc0 = acc0 + u0 * v00 + u1 * v01
                acc1 = acc1 + u0 * v10 + u1 * v11
            b0_v[pl.ds(e0, 16)] = acc0
            b1_v[pl.ds(e0, 16)] = acc1
            return c2

        lax.fori_loop(0, ECH // 16, grp, 0)
        pltpu.sync_copy(b0_v, bout_hbm.at[pl.ds(off, ECH)])
        pltpu.sync_copy(b1_v, bout_hbm.at[pl.ds(E_PAD + off, ECH)])


@functools.cache
def _sc_decoder_fn():
    return pl.kernel(
        _sc_decoder_body,
        out_type=jax.ShapeDtypeStruct((2 * E_PAD,), F32),
        mesh=plsc.VectorSubcoreMesh(core_axis_name="c", subcore_axis_name="s"),
        compiler_params=pltpu.CompilerParams(needs_layout_passes=False),
        scratch_types=[
            pltpu.VMEM((NU * H1 // 2,), jnp.int32),
            pltpu.VMEM((NI * H1,), jnp.int32),
            pltpu.VMEM((ECH,), jnp.int32),
            pltpu.VMEM((ECH,), jnp.int32),
            pltpu.VMEM((ECH,), F32),
            pltpu.VMEM((ECH,), F32),
        ],
    )


def _sc_decoder(up, vp, uht_flat, vbp_flat):
    bcat = _sc_decoder_fn()(up, vp, uht_flat, vbp_flat)
    return bcat[:E_PAD], bcat[E_PAD:]


def _head_kernel(b0_ref, b1_ref, r_ref, amix_ref, mh_ref, loss_ref, rmse_ref):
    b0 = b0_ref[...]
    b1 = b1_ref[...]
    r = r_ref[...]
    o = [b0 * amix_ref[0, k] + b1 * amix_ref[0, NC + k] for k in range(NC)]
    mx = o[0]
    for k in range(1, NC):
        mx = jnp.maximum(mx, o[k])
    e = [jnp.exp(o[k] - mx) for k in range(NC)]
    s = e[0]
    num = e[0]
    for k in range(1, NC):
        s = s + e[k]
        num = num + e[k] * (k + 1.0)
    mh = num / s
    mh_ref[...] = mh

    rows = b0.shape[0]
    cols = b0.shape[1]
    eid = (lax.broadcasted_iota(jnp.int32, (rows, cols), 0) * cols
           + lax.broadcasted_iota(jnp.int32, (rows, cols), 1))
    valid = eid < NE
    o_r = jnp.zeros_like(b0)
    for k in range(NC):
        o_r = o_r + jnp.where(r == k, o[k], 0.0)
    logp_r = o_r - mx - jnp.log(s)
    loss = -jnp.sum(jnp.where(valid, logp_r, 0.0)) / NE
    sq = jnp.where(valid, (mh - (r.astype(F32) + 1.0)) ** 2, 0.0)
    rmse = jnp.sqrt(jnp.sum(sq) / NE)
    loss_ref[0, 0] = loss
    rmse_ref[0, 0] = rmse


def _head(b0, b1, r2d, amix):
    rows, cols = b0.shape
    return pl.pallas_call(
        _head_kernel,
        in_specs=[
            pl.BlockSpec((rows, cols), lambda: (0, 0)),
            pl.BlockSpec((rows, cols), lambda: (0, 0)),
            pl.BlockSpec((rows, cols), lambda: (0, 0)),
            pl.BlockSpec((1, 2 * NC), lambda: (0, 0)),
        ],
        out_specs=[
            pl.BlockSpec((rows, cols), lambda: (0, 0)),
            pl.BlockSpec(memory_space=pltpu.SMEM),
            pl.BlockSpec(memory_space=pltpu.SMEM),
        ],
        out_shape=[
            jax.ShapeDtypeStruct((rows, cols), F32),
            jax.ShapeDtypeStruct((1, 1), F32),
            jax.ShapeDtypeStruct((1, 1), F32),
        ],
    )(b0, b1, r2d, amix)


def kernel(u, v, r, support, support_t, u_side, v_side, u_features,
           v_features, W_gc, W_u1, b_u1, W_v1, b_v1, W_u2, W_v2, W_bil,
           a_mix):
    tmp_u = _feat_matmul(u_features, W_gc)
    tmp_v = _feat_matmul(v_features, W_gc)

    wbT = jnp.concatenate([W_bil[0].T, W_bil[1].T], axis=1)
    u_h = _conv_side(support, tmp_v, u_side, W_u1, b_u1.reshape(1, ID),
                     W_u2, wbT, emit_basis=False)
    vb = _conv_side(support_t, tmp_u, v_side, W_v1, b_v1.reshape(1, ID),
                    W_v2, wbT, emit_basis=True)
    u3 = u_h.T.reshape(H1 // 2, 2, NU).transpose(0, 2, 1)
    uht_flat = lax.bitcast_convert_type(
        u3.astype(jnp.bfloat16), jnp.int32).reshape(H1 // 2 * NU)
    vb3 = jnp.stack([vb[:, :H1].T, vb[:, H1:].T], axis=-1)
    vbp_flat = lax.bitcast_convert_type(
        vb3.astype(jnp.bfloat16), jnp.int32).reshape(H1 * NI)

    pad = E_PAD - NE
    up = jnp.pad(u.astype(jnp.int32), (0, pad))
    vp = jnp.pad(v.astype(jnp.int32), (0, pad))
    rp = jnp.pad(r.astype(jnp.int32), (0, pad))

    b0, b1 = _sc_decoder(up, vp, uht_flat, vbp_flat)

    mh2, loss11, rmse11 = _head(b0.reshape(784, 128), b1.reshape(784, 128),
                                rp.reshape(784, 128), a_mix.reshape(1, 2 * NC))
    m_hat = mh2.reshape(E_PAD)[:NE]
    return (m_hat, loss11[0, 0], rmse11[0, 0])
